# Initial kernel scaffold; baseline (speedup 1.0000x reference)
import jax
import jax.numpy as jnp
from jax.experimental import pallas as pl


def kernel(fluid_node_attr, env_node_attr, mesh_edge_attr, world_edge_attr, params, mesh_edge_index, world_edge_index):
    raise NotImplementedError("write your pallas kernel here")


# speedup vs baseline: 2.2541x; 2.2541x over previous
import functools

import jax
import jax.numpy as jnp
from jax import lax
from jax.experimental import pallas as pl
from jax.experimental.pallas import tpu as pltpu
from jax.experimental.pallas import tpu_sc as plsc

L = 32
NF = 50000
NE = 10000
EM = 800000
EW = 200000
OUT = 3

NC = 2
NS = 16
NW = NC * NS
BLK = 128

NB_M = 196
NB_W = 49
EMP = NW * NB_M * BLK
EWP = NW * NB_W * BLK
NB_ALL = 2 * NB_M + NB_W
GM = NW * NB_ALL * BLK

NFP = 50048
NEP = 10048
STRIPE = NFP // NS

ROW_T = 4096


def _pad_rows(x, n):
    return jnp.pad(x, ((0, n - x.shape[0]), (0, 0)))


def _pad_idx(x, n, fill):
    return jnp.pad(x, (0, n - x.shape[0]), constant_values=fill)


def _ln(x, g, b):
    m = jnp.mean(x, axis=-1, keepdims=True)
    v = jnp.mean((x - m) * (x - m), axis=-1, keepdims=True)
    return (x - m) * lax.rsqrt(v + 1e-5) * g + b


def _dot(a, b):
    return jnp.dot(a, b, preferred_element_type=jnp.float32)


def _pack_vecs(p, with_ln):
    mlp = p["mlp"]
    rows = []
    for i in range(3):
        b = mlp[i]["b"]
        rows.append(jnp.pad(b, (0, L - b.shape[0])))
    if with_ln:
        rows.append(p["ln_g"])
        rows.append(p["ln_b"])
    else:
        rows.append(jnp.zeros((L,), jnp.float32))
        rows.append(jnp.zeros((L,), jnp.float32))
    rows += [jnp.zeros((L,), jnp.float32)] * 3
    return jnp.stack(rows)


def _enc_body(x_ref, w1_ref, w2_ref, w3_ref, v_ref, o_ref):
    x = _dot(x_ref[...], w1_ref[...]) + v_ref[0:1, :]
    x = jnp.maximum(x, 0.0)
    x = jnp.maximum(_dot(x, w2_ref[...]) + v_ref[1:2, :], 0.0)
    x = _dot(x, w3_ref[...]) + v_ref[2:3, :]
    o_ref[...] = _ln(x, v_ref[3:4, :], v_ref[4:5, :])


def _encoder(p, x, blk):
    n, f = x.shape
    mlp = p["mlp"]
    full = lambda s: pl.BlockSpec(s, lambda i: (0,) * len(s))
    return pl.pallas_call(
        _enc_body,
        grid=(n // blk,),
        in_specs=[
            pl.BlockSpec((blk, f), lambda i: (i, 0)),
            full((f, L)), full((L, L)), full((L, L)), full((8, L)),
        ],
        out_specs=pl.BlockSpec((blk, L), lambda i: (i, 0)),
        out_shape=jax.ShapeDtypeStruct((n, L), jnp.float32),
        compiler_params=pltpu.CompilerParams(
            dimension_semantics=("arbitrary",)),
    )(x, mlp[0]["W"], mlp[1]["W"], mlp[2]["W"], _pack_vecs(p, True))


def _edge_body(e_ref, s_ref, d_ref, w1_ref, w2_ref, w3_ref, v_ref,
               new_ref, res_ref):
    e = e_ref[...]
    x = (_dot(e, w1_ref[0:L, :])
         + _dot(s_ref[...], w1_ref[L:2 * L, :])
         + _dot(d_ref[...], w1_ref[2 * L:3 * L, :])
         + v_ref[0:1, :])
    x = jnp.maximum(x, 0.0)
    x = jnp.maximum(_dot(x, w2_ref[...]) + v_ref[1:2, :], 0.0)
    x = _dot(x, w3_ref[...]) + v_ref[2:3, :]
    x = _ln(x, v_ref[3:4, :], v_ref[4:5, :])
    new_ref[...] = x
    res_ref[...] = e + x


def _edge_mlp(p, e, src, dst):
    n = e.shape[0]
    grid = n // ROW_T
    mlp = p["mlp"]
    full = lambda s: pl.BlockSpec(s, lambda i: (0, 0))
    (sa, so), (da, do) = src, dst
    return pl.pallas_call(
        _edge_body,
        grid=(grid,),
        in_specs=[
            pl.BlockSpec((ROW_T, L), lambda i: (i, 0)),
            pl.BlockSpec((ROW_T, L), lambda i, _o=so: (i + _o, 0)),
            pl.BlockSpec((ROW_T, L), lambda i, _o=do: (i + _o, 0)),
            full((3 * L, L)), full((L, L)), full((L, L)), full((8, L)),
        ],
        out_specs=[pl.BlockSpec((ROW_T, L), lambda i: (i, 0))] * 2,
        out_shape=[jax.ShapeDtypeStruct((n, L), jnp.float32)] * 2,
        compiler_params=pltpu.CompilerParams(
            dimension_semantics=("arbitrary",)),
    )(e, sa, da, mlp[0]["W"], mlp[1]["W"], mlp[2]["W"], _pack_vecs(p, True))


def _node_body(f_ref, am_ref, aw_ref, w1_ref, w2_ref, w3_ref, v_ref, o_ref):
    f = f_ref[...]
    am = am_ref[0] + am_ref[1]
    aw = aw_ref[0] + aw_ref[1]
    x = (_dot(f, w1_ref[0:L, :])
         + _dot(am, w1_ref[L:2 * L, :])
         + _dot(aw, w1_ref[2 * L:3 * L, :])
         + v_ref[0:1, :])
    x = jnp.maximum(x, 0.0)
    x = jnp.maximum(_dot(x, w2_ref[...]) + v_ref[1:2, :], 0.0)
    x = _dot(x, w3_ref[...]) + v_ref[2:3, :]
    o_ref[...] = f + _ln(x, v_ref[3:4, :], v_ref[4:5, :])


def _node_mlp(p, fl, amp, awp):
    blk = 3128
    grid = NFP // blk
    mlp = p["mlp"]
    full = lambda s: pl.BlockSpec(s, lambda i: (0,) * len(s))
    return pl.pallas_call(
        _node_body,
        grid=(grid,),
        in_specs=[
            pl.BlockSpec((blk, L), lambda i: (i, 0)),
            pl.BlockSpec((2, blk, L), lambda i: (0, i, 0)),
            pl.BlockSpec((2, blk, L), lambda i: (0, i, 0)),
            full((3 * L, L)), full((L, L)), full((L, L)), full((8, L)),
        ],
        out_specs=pl.BlockSpec((blk, L), lambda i: (i, 0)),
        out_shape=jax.ShapeDtypeStruct((NFP, L), jnp.float32),
        compiler_params=pltpu.CompilerParams(
            dimension_semantics=("arbitrary",)),
    )(fl, amp, awp, mlp[0]["W"], mlp[1]["W"], mlp[2]["W"], _pack_vecs(p, True))


def _dec_body(x_ref, w1_ref, w2_ref, w3_ref, v_ref, o_ref):
    x = jnp.maximum(_dot(x_ref[...], w1_ref[...]) + v_ref[0:1, :], 0.0)
    x = jnp.maximum(_dot(x, w2_ref[...]) + v_ref[1:2, :], 0.0)
    o_ref[...] = _dot(x, w3_ref[...]) + v_ref[2:3, 0:OUT]


def _decoder(p, fl):
    blk = 3128
    mlp = p["mlp"]
    full = lambda s: pl.BlockSpec(s, lambda i: (0, 0))
    return pl.pallas_call(
        _dec_body,
        grid=(NFP // blk,),
        in_specs=[
            pl.BlockSpec((blk, L), lambda i: (i, 0)),
            full((L, L)), full((L, L)), full((L, OUT)), full((8, L)),
        ],
        out_specs=pl.BlockSpec((blk, OUT), lambda i: (i, 0)),
        out_shape=jax.ShapeDtypeStruct((NFP, OUT), jnp.float32),
        compiler_params=pltpu.CompilerParams(
            dimension_semantics=("arbitrary",)),
    )(fl, mlp[0]["W"], mlp[1]["W"], mlp[2]["W"], _pack_vecs(p, False))


@functools.cache
def _sc_mesh():
    return plsc.VectorSubcoreMesh(
        core_axis_name="c", subcore_axis_name="s",
        num_cores=NC, num_subcores=NS)


def _gather_call(table, idx3, nb):
    rows_pw = nb * BLK
    n = NW * rows_pw

    @functools.partial(
        pl.kernel,
        out_type=jax.ShapeDtypeStruct((n, L), jnp.float32),
        mesh=_sc_mesh(),
        compiler_params=pltpu.CompilerParams(use_tc_tiling_on_sc=False),
        scratch_types=[
            pltpu.VMEM((nb, BLK), jnp.int32),
            pltpu.VMEM((BLK, L), jnp.float32),
            pltpu.VMEM((BLK, L), jnp.float32),
            pltpu.SemaphoreType.DMA,
            pltpu.SemaphoreType.DMA,
        ],
    )
    def k(table_ref, idx_ref, out_ref, idx_v, buf0, buf1, sem0, sem1):
        wid = lax.axis_index("s") * NC + lax.axis_index("c")
        base = wid * rows_pw
        pltpu.sync_copy(idx_ref.at[wid], idx_v)

        def gather(b, buf, sem):
            return pltpu.async_copy(table_ref.at[idx_v.at[b]], buf, sem)

        def store(b, buf, sem):
            return pltpu.async_copy(
                buf, out_ref.at[pl.ds(base + b * BLK, BLK)], sem)

        gather(0, buf0, sem0).wait()
        st_prev = store(0, buf0, sem0)

        def body(b, _):
            g = gather(b, buf1, sem1)
            g.wait()
            st = store(b, buf1, sem1)
            st.wait()
            return 0

        lax.fori_loop(1, nb, body, 0)
        st_prev.wait()

    return k(table, idx3)


def _scatter_call(vals, idx3, zeros, nb):
    rows_pw = nb * BLK

    @functools.partial(
        pl.kernel,
        out_type=jax.ShapeDtypeStruct((NC, NFP, L), jnp.float32),
        mesh=_sc_mesh(),
        compiler_params=pltpu.CompilerParams(use_tc_tiling_on_sc=False),
        scratch_types=[
            pltpu.VMEM((nb, BLK), jnp.int32),
            pltpu.VMEM((BLK, L), jnp.float32),
            pltpu.VMEM_SHARED((NFP, L), jnp.float32),
        ],
    )
    def k(vals_ref, idx_ref, z_ref, out_ref, idx_v, buf, acc):
        c = lax.axis_index("c")
        s = lax.axis_index("s")
        wid = s * NC + c
        base = wid * rows_pw
        pltpu.sync_copy(z_ref.at[pl.ds(s * STRIPE, STRIPE)],
                        acc.at[pl.ds(s * STRIPE, STRIPE)])
        plsc.subcore_barrier()

        def body(b, _):
            pltpu.sync_copy(vals_ref.at[pl.ds(base + b * BLK, BLK)], buf)
            pltpu.sync_copy(buf, acc.at[idx_v.at[b]], add=True)
            return 0

        pltpu.sync_copy(idx_ref.at[wid], idx_v)
        lax.fori_loop(0, nb, body, 0)
        plsc.subcore_barrier()
        pltpu.sync_copy(acc.at[pl.ds(s * STRIPE, STRIPE)],
                        out_ref.at[c, pl.ds(s * STRIPE, STRIPE)])

    return k(vals, idx3, zeros)


def kernel(fluid_node_attr, env_node_attr, mesh_edge_attr, world_edge_attr,
           params, mesh_edge_index, world_edge_index):
    p = params
    fl_attr = _pad_rows(fluid_node_attr, NFP)
    env_attr = _pad_rows(env_node_attr, NEP)
    me_attr = _pad_rows(mesh_edge_attr, EMP)
    we_attr = _pad_rows(world_edge_attr, EWP)

    ms = _pad_idx(mesh_edge_index[0], EMP, 0)
    md = _pad_idx(mesh_edge_index[1], EMP, NFP - 1)
    ws = _pad_idx(world_edge_index[0], EWP, 0)
    wd = _pad_idx(world_edge_index[1], EWP, NFP - 1)
    gidx = jnp.concatenate([ms, md, wd]).reshape(NW, NB_ALL, BLK)
    ws3 = ws.reshape(NW, NB_W, BLK)
    md3 = md.reshape(NW, NB_M, BLK)
    wd3 = wd.reshape(NW, NB_W, BLK)
    zeros_nf = jnp.zeros((NFP, L), jnp.float32)

    fl = _encoder(p["node_enc"], fl_attr, 3128)
    el = _encoder(p["node_enc"], env_attr, 2512)
    me = _encoder(p["mesh_enc"], me_attr, ROW_T)
    we = _encoder(p["world_enc"], we_attr, ROW_T)

    gws = _gather_call(el, ws3, NB_W)

    nbm = EMP // ROW_T
    for sp in p["steps"]:
        g = _gather_call(fl, gidx, NB_ALL)
        mnew, me = _edge_mlp(sp["mesh_edge"], me, (g, 0), (g, nbm))
        wnew, we = _edge_mlp(sp["world_edge"], we, (gws, 0), (g, 2 * nbm))
        amp = _scatter_call(mnew, md3, zeros_nf, NB_M)
        awp = _scatter_call(wnew, wd3, zeros_nf, NB_W)
        fl = _node_mlp(sp["node"], fl, amp, awp)

    return _decoder(p["decoder"], fl)[:NF]


# speedup vs baseline: 2.4458x; 1.0850x over previous
import functools

import jax
import jax.numpy as jnp
from jax import lax
from jax.experimental import pallas as pl
from jax.experimental.pallas import tpu as pltpu
from jax.experimental.pallas import tpu_sc as plsc

L = 32
NF = 50000
NE = 10000
EM = 800000
EW = 200000
OUT = 3

NC = 2
NS = 16
NW = NC * NS
BLK = 128

NB_M = 196
NB_W = 49
EMP = NW * NB_M * BLK
EWP = NW * NB_W * BLK
NB_ALL = 2 * NB_M + NB_W
GM = NW * NB_ALL * BLK

NFP = 50048
NEP = 10048
STRIPE = NFP // NS

ROW_T = 4096


def _pad_rows(x, n):
    return jnp.pad(x, ((0, n - x.shape[0]), (0, 0)))


def _pad_idx(x, n, fill):
    return jnp.pad(x, (0, n - x.shape[0]), constant_values=fill)


def _ln(x, g, b):
    m = jnp.mean(x, axis=-1, keepdims=True)
    v = jnp.mean((x - m) * (x - m), axis=-1, keepdims=True)
    return (x - m) * lax.rsqrt(v + 1e-5) * g + b


def _dot(a, b):
    return jnp.dot(a, b, preferred_element_type=jnp.float32)


def _pack_vecs(p, with_ln):
    mlp = p["mlp"]
    rows = []
    for i in range(3):
        b = mlp[i]["b"]
        rows.append(jnp.pad(b, (0, L - b.shape[0])))
    if with_ln:
        rows.append(p["ln_g"])
        rows.append(p["ln_b"])
    else:
        rows.append(jnp.zeros((L,), jnp.float32))
        rows.append(jnp.zeros((L,), jnp.float32))
    rows += [jnp.zeros((L,), jnp.float32)] * 3
    return jnp.stack(rows)


def _enc_body(x_ref, w1_ref, w2_ref, w3_ref, v_ref, o_ref):
    x = _dot(x_ref[...], w1_ref[...]) + v_ref[0:1, :]
    x = jnp.maximum(x, 0.0)
    x = jnp.maximum(_dot(x, w2_ref[...]) + v_ref[1:2, :], 0.0)
    x = _dot(x, w3_ref[...]) + v_ref[2:3, :]
    o_ref[...] = _ln(x, v_ref[3:4, :], v_ref[4:5, :])


def _encoder(p, x, blk):
    n, f = x.shape
    mlp = p["mlp"]
    full = lambda s: pl.BlockSpec(s, lambda i: (0,) * len(s))
    return pl.pallas_call(
        _enc_body,
        grid=(n // blk,),
        in_specs=[
            pl.BlockSpec((blk, f), lambda i: (i, 0)),
            full((f, L)), full((L, L)), full((L, L)), full((8, L)),
        ],
        out_specs=pl.BlockSpec((blk, L), lambda i: (i, 0)),
        out_shape=jax.ShapeDtypeStruct((n, L), jnp.float32),
        compiler_params=pltpu.CompilerParams(
            dimension_semantics=("arbitrary",)),
    )(x, mlp[0]["W"], mlp[1]["W"], mlp[2]["W"], _pack_vecs(p, True))


def _edge_body(e_ref, s_ref, d_ref, w1_ref, w2_ref, w3_ref, v_ref,
               new_ref, res_ref):
    e = e_ref[...]
    x = (_dot(e, w1_ref[0:L, :])
         + _dot(s_ref[...], w1_ref[L:2 * L, :])
         + _dot(d_ref[...], w1_ref[2 * L:3 * L, :])
         + v_ref[0:1, :])
    x = jnp.maximum(x, 0.0)
    x = jnp.maximum(_dot(x, w2_ref[...]) + v_ref[1:2, :], 0.0)
    x = _dot(x, w3_ref[...]) + v_ref[2:3, :]
    x = _ln(x, v_ref[3:4, :], v_ref[4:5, :])
    new_ref[...] = x
    res_ref[...] = e + x


def _edge_mlp(p, e, src, dst):
    n = e.shape[0]
    grid = n // ROW_T
    mlp = p["mlp"]
    full = lambda s: pl.BlockSpec(s, lambda i: (0, 0))
    (sa, so), (da, do) = src, dst
    return pl.pallas_call(
        _edge_body,
        grid=(grid,),
        in_specs=[
            pl.BlockSpec((ROW_T, L), lambda i: (i, 0)),
            pl.BlockSpec((ROW_T, L), lambda i, _o=so: (i + _o, 0)),
            pl.BlockSpec((ROW_T, L), lambda i, _o=do: (i + _o, 0)),
            full((3 * L, L)), full((L, L)), full((L, L)), full((8, L)),
        ],
        out_specs=[pl.BlockSpec((ROW_T, L), lambda i: (i, 0))] * 2,
        out_shape=[jax.ShapeDtypeStruct((n, L), jnp.float32)] * 2,
        compiler_params=pltpu.CompilerParams(
            dimension_semantics=("arbitrary",)),
    )(e, sa, da, mlp[0]["W"], mlp[1]["W"], mlp[2]["W"], _pack_vecs(p, True))


def _node_body(f_ref, am_ref, aw_ref, w1_ref, w2_ref, w3_ref, v_ref, o_ref):
    f = f_ref[...]
    am = am_ref[0] + am_ref[1]
    aw = aw_ref[0] + aw_ref[1]
    x = (_dot(f, w1_ref[0:L, :])
         + _dot(am, w1_ref[L:2 * L, :])
         + _dot(aw, w1_ref[2 * L:3 * L, :])
         + v_ref[0:1, :])
    x = jnp.maximum(x, 0.0)
    x = jnp.maximum(_dot(x, w2_ref[...]) + v_ref[1:2, :], 0.0)
    x = _dot(x, w3_ref[...]) + v_ref[2:3, :]
    o_ref[...] = f + _ln(x, v_ref[3:4, :], v_ref[4:5, :])


def _node_mlp(p, fl, amp, awp):
    blk = 3128
    grid = NFP // blk
    mlp = p["mlp"]
    full = lambda s: pl.BlockSpec(s, lambda i: (0,) * len(s))
    return pl.pallas_call(
        _node_body,
        grid=(grid,),
        in_specs=[
            pl.BlockSpec((blk, L), lambda i: (i, 0)),
            pl.BlockSpec((2, blk, L), lambda i: (0, i, 0)),
            pl.BlockSpec((2, blk, L), lambda i: (0, i, 0)),
            full((3 * L, L)), full((L, L)), full((L, L)), full((8, L)),
        ],
        out_specs=pl.BlockSpec((blk, L), lambda i: (i, 0)),
        out_shape=jax.ShapeDtypeStruct((NFP, L), jnp.float32),
        compiler_params=pltpu.CompilerParams(
            dimension_semantics=("arbitrary",)),
    )(fl, amp, awp, mlp[0]["W"], mlp[1]["W"], mlp[2]["W"], _pack_vecs(p, True))


def _dec_body(x_ref, w1_ref, w2_ref, w3_ref, v_ref, o_ref):
    x = jnp.maximum(_dot(x_ref[...], w1_ref[...]) + v_ref[0:1, :], 0.0)
    x = jnp.maximum(_dot(x, w2_ref[...]) + v_ref[1:2, :], 0.0)
    o_ref[...] = _dot(x, w3_ref[...]) + v_ref[2:3, 0:OUT]


def _decoder(p, fl):
    blk = 3128
    mlp = p["mlp"]
    full = lambda s: pl.BlockSpec(s, lambda i: (0, 0))
    return pl.pallas_call(
        _dec_body,
        grid=(NFP // blk,),
        in_specs=[
            pl.BlockSpec((blk, L), lambda i: (i, 0)),
            full((L, L)), full((L, L)), full((L, OUT)), full((8, L)),
        ],
        out_specs=pl.BlockSpec((blk, OUT), lambda i: (i, 0)),
        out_shape=jax.ShapeDtypeStruct((NFP, OUT), jnp.float32),
        compiler_params=pltpu.CompilerParams(
            dimension_semantics=("arbitrary",)),
    )(fl, mlp[0]["W"], mlp[1]["W"], mlp[2]["W"], _pack_vecs(p, False))


@functools.cache
def _sc_mesh():
    return plsc.VectorSubcoreMesh(
        core_axis_name="c", subcore_axis_name="s",
        num_cores=NC, num_subcores=NS)


NK = 7


def _gather_call(table, idx3, nb):
    rows_pw = nb * BLK
    n = NW * rows_pw
    ng = nb // NK

    @functools.partial(
        pl.kernel,
        out_type=jax.ShapeDtypeStruct((n, L), jnp.float32),
        mesh=_sc_mesh(),
        compiler_params=pltpu.CompilerParams(use_tc_tiling_on_sc=False),
        scratch_types=[
            pltpu.VMEM((nb, BLK), jnp.int32),
            pltpu.VMEM((NK, BLK, L), jnp.float32),
            pltpu.SemaphoreType.DMA,
            pltpu.SemaphoreType.DMA,
        ],
    )
    def k(table_ref, idx_ref, out_ref, idx_v, bufs, semg, sems):
        wid = lax.axis_index("s") * NC + lax.axis_index("c")
        base = wid * rows_pw
        pltpu.sync_copy(idx_ref.at[wid], idx_v)

        def body(g, _):
            b0 = g * NK
            ds = [pltpu.async_copy(table_ref.at[idx_v.at[b0 + j]],
                                   bufs.at[j], semg) for j in range(NK)]
            for d in ds:
                d.wait()
            ss = [pltpu.async_copy(
                bufs.at[j],
                out_ref.at[pl.ds(base + (b0 + j) * BLK, BLK)], sems)
                for j in range(NK)]
            for d in ss:
                d.wait()
            return 0

        lax.fori_loop(0, ng, body, 0)

    return k(table, idx3)


def _scatter_call(vals, idx3, zeros, nb):
    rows_pw = nb * BLK

    @functools.partial(
        pl.kernel,
        out_type=jax.ShapeDtypeStruct((NC, NFP, L), jnp.float32),
        mesh=_sc_mesh(),
        compiler_params=pltpu.CompilerParams(use_tc_tiling_on_sc=False),
        scratch_types=[
            pltpu.VMEM((NK, BLK), jnp.int32),
            pltpu.VMEM((NK, BLK, L), jnp.float32),
            pltpu.VMEM_SHARED((NFP, L), jnp.float32),
            pltpu.SemaphoreType.DMA,
            pltpu.SemaphoreType.DMA,
        ],
    )
    def k(vals_ref, idx_ref, z_ref, out_ref, idx_v, bufs, acc, semr, semw):
        c = lax.axis_index("c")
        s = lax.axis_index("s")
        wid = s * NC + c
        base = wid * rows_pw
        ng = nb // NK
        pltpu.sync_copy(z_ref.at[pl.ds(s * STRIPE, STRIPE)],
                        acc.at[pl.ds(s * STRIPE, STRIPE)])
        plsc.subcore_barrier()

        def body(g, _):
            b0 = g * NK
            di = pltpu.async_copy(idx_ref.at[wid, pl.ds(b0, NK)], idx_v,
                                  semr)
            ds = [pltpu.async_copy(
                vals_ref.at[pl.ds(base + (b0 + j) * BLK, BLK)],
                bufs.at[j], semr) for j in range(NK)]
            di.wait()
            for d in ds:
                d.wait()
            ss = [pltpu.async_copy(bufs.at[j], acc.at[idx_v.at[j]],
                                   semw, add=True) for j in range(NK)]
            for d in ss:
                d.wait()
            return 0

        lax.fori_loop(0, ng, body, 0)
        plsc.subcore_barrier()
        pltpu.sync_copy(acc.at[pl.ds(s * STRIPE, STRIPE)],
                        out_ref.at[c, pl.ds(s * STRIPE, STRIPE)])

    return k(vals, idx3, zeros)


def kernel(fluid_node_attr, env_node_attr, mesh_edge_attr, world_edge_attr,
           params, mesh_edge_index, world_edge_index):
    p = params
    fl_attr = _pad_rows(fluid_node_attr, NFP)
    env_attr = _pad_rows(env_node_attr, NEP)
    me_attr = _pad_rows(mesh_edge_attr, EMP)
    we_attr = _pad_rows(world_edge_attr, EWP)

    ms = _pad_idx(mesh_edge_index[0], EMP, 0)
    md = _pad_idx(mesh_edge_index[1], EMP, NFP - 1)
    ws = _pad_idx(world_edge_index[0], EWP, 0)
    wd = _pad_idx(world_edge_index[1], EWP, NFP - 1)
    gidx = jnp.concatenate([ms, md, wd]).reshape(NW, NB_ALL, BLK)
    ws3 = ws.reshape(NW, NB_W, BLK)
    md3 = md.reshape(NW, NB_M, BLK)
    wd3 = wd.reshape(NW, NB_W, BLK)
    zeros_nf = jnp.zeros((NFP, L), jnp.float32)

    fl = _encoder(p["node_enc"], fl_attr, 3128)
    el = _encoder(p["node_enc"], env_attr, 2512)
    me = _encoder(p["mesh_enc"], me_attr, ROW_T)
    we = _encoder(p["world_enc"], we_attr, ROW_T)

    gws = _gather_call(el, ws3, NB_W)

    nbm = EMP // ROW_T
    for sp in p["steps"]:
        g = _gather_call(fl, gidx, NB_ALL)
        mnew, me = _edge_mlp(sp["mesh_edge"], me, (g, 0), (g, nbm))
        wnew, we = _edge_mlp(sp["world_edge"], we, (gws, 0), (g, 2 * nbm))
        amp = _scatter_call(mnew, md3, zeros_nf, NB_M)
        awp = _scatter_call(wnew, wd3, zeros_nf, NB_W)
        fl = _node_mlp(sp["node"], fl, amp, awp)

    return _decoder(p["decoder"], fl)[:NF]


# speedup vs baseline: 6.1601x; 2.5187x over previous
import functools

import jax
import jax.numpy as jnp
from jax import lax
from jax.experimental import pallas as pl
from jax.experimental.pallas import tpu as pltpu
from jax.experimental.pallas import tpu_sc as plsc

L = 32
NF = 50000
NE = 10000
EM = 800000
EW = 200000
OUT = 3

NC = 2
NS = 16
NW = NC * NS
BLK = 128

NB_M = 196
NB_W = 49
EMP = NW * NB_M * BLK
EWP = NW * NB_W * BLK
NB_ALL = 2 * NB_M + NB_W
GM = NW * NB_ALL * BLK

NFP = 50048
NEP = 10048
STRIPE = NFP // NS

ROW_T = 4096


def _pad_rows(x, n):
    return jnp.pad(x, ((0, n - x.shape[0]), (0, 0)))


def _pad_idx(x, n, fill):
    return jnp.pad(x, (0, n - x.shape[0]), constant_values=fill)


def _ln(x, g, b):
    m = jnp.mean(x, axis=-1, keepdims=True)
    v = jnp.mean((x - m) * (x - m), axis=-1, keepdims=True)
    return (x - m) * lax.rsqrt(v + 1e-5) * g + b


def _dot(a, b):
    return jnp.dot(a, b, preferred_element_type=jnp.float32)


def _pack_vecs(p, with_ln):
    mlp = p["mlp"]
    rows = []
    for i in range(3):
        b = mlp[i]["b"]
        rows.append(jnp.pad(b, (0, L - b.shape[0])))
    if with_ln:
        rows.append(p["ln_g"])
        rows.append(p["ln_b"])
    else:
        rows.append(jnp.zeros((L,), jnp.float32))
        rows.append(jnp.zeros((L,), jnp.float32))
    rows += [jnp.zeros((L,), jnp.float32)] * 3
    return jnp.stack(rows)


def _enc_body(x_ref, w1_ref, w2_ref, w3_ref, v_ref, o_ref):
    x = _dot(x_ref[...], w1_ref[...]) + v_ref[0:1, :]
    x = jnp.maximum(x, 0.0)
    x = jnp.maximum(_dot(x, w2_ref[...]) + v_ref[1:2, :], 0.0)
    x = _dot(x, w3_ref[...]) + v_ref[2:3, :]
    o_ref[...] = _ln(x, v_ref[3:4, :], v_ref[4:5, :])


def _encoder(p, x, blk):
    n, f = x.shape
    mlp = p["mlp"]
    full = lambda s: pl.BlockSpec(s, lambda i: (0,) * len(s))
    return pl.pallas_call(
        _enc_body,
        grid=(n // blk,),
        in_specs=[
            pl.BlockSpec((blk, f), lambda i: (i, 0)),
            full((f, L)), full((L, L)), full((L, L)), full((8, L)),
        ],
        out_specs=pl.BlockSpec((blk, L), lambda i: (i, 0)),
        out_shape=jax.ShapeDtypeStruct((n, L), jnp.float32),
        compiler_params=pltpu.CompilerParams(
            dimension_semantics=("arbitrary",)),
    )(x, mlp[0]["W"], mlp[1]["W"], mlp[2]["W"], _pack_vecs(p, True))


LP = 4 * L
RT_P = ROW_T // 4


def _blk4(w):
    return jnp.kron(jnp.eye(4, dtype=jnp.float32), w)


def _mean_mat():
    return _blk4(jnp.full((L, L), 1.0 / L, jnp.float32))


def _pack_vecs_p(p):
    mlp = p["mlp"]
    rows = [jnp.tile(mlp[i]["b"], 4) for i in range(3)]
    rows.append(jnp.tile(p["ln_g"], 4))
    rows.append(jnp.tile(p["ln_b"], 4))
    rows += [jnp.zeros((LP,), jnp.float32)] * 3
    return jnp.stack(rows)


def _ln_p(x, mb, g, b):
    m = _dot(x, mb)
    d = x - m
    v = _dot(d * d, mb)
    return d * lax.rsqrt(v + 1e-5) * g + b


def _edge_body_p(e_ref, s_ref, d_ref, w1a_ref, w1b_ref, w1c_ref,
                 w2_ref, w3_ref, mb_ref, v_ref, new_ref, res_ref):
    e = e_ref[...]
    x = (_dot(e, w1a_ref[...])
         + _dot(s_ref[...], w1b_ref[...])
         + _dot(d_ref[...], w1c_ref[...])
         + v_ref[0:1, :])
    x = jnp.maximum(x, 0.0)
    x = jnp.maximum(_dot(x, w2_ref[...]) + v_ref[1:2, :], 0.0)
    x = _dot(x, w3_ref[...]) + v_ref[2:3, :]
    x = _ln_p(x, mb_ref[...], v_ref[3:4, :], v_ref[4:5, :])
    new_ref[...] = x
    res_ref[...] = e + x


def _edge_mlp(p, e, src, dst, mb):
    n = e.shape[0]
    grid = n // RT_P
    mlp = p["mlp"]
    full = lambda s: pl.BlockSpec(s, lambda i: (0, 0))
    (sa, so), (da, do) = src, dst
    w1 = mlp[0]["W"]
    return pl.pallas_call(
        _edge_body_p,
        grid=(grid,),
        in_specs=[
            pl.BlockSpec((RT_P, LP), lambda i: (i, 0)),
            pl.BlockSpec((RT_P, LP), lambda i, _o=so: (i + _o, 0)),
            pl.BlockSpec((RT_P, LP), lambda i, _o=do: (i + _o, 0)),
            full((LP, LP)), full((LP, LP)), full((LP, LP)),
            full((LP, LP)), full((LP, LP)), full((LP, LP)), full((8, LP)),
        ],
        out_specs=[pl.BlockSpec((RT_P, LP), lambda i: (i, 0))] * 2,
        out_shape=[jax.ShapeDtypeStruct((n, LP), jnp.float32)] * 2,
        compiler_params=pltpu.CompilerParams(
            dimension_semantics=("arbitrary",)),
    )(e, sa, da, _blk4(w1[0:L]), _blk4(w1[L:2 * L]), _blk4(w1[2 * L:]),
      _blk4(mlp[1]["W"]), _blk4(mlp[2]["W"]), mb, _pack_vecs_p(p))


def _node_body_p(f_ref, am_ref, aw_ref, w1a_ref, w1b_ref, w1c_ref,
                 w2_ref, w3_ref, mb_ref, v_ref, o_ref):
    f = f_ref[...]
    x = (_dot(f, w1a_ref[...])
         + _dot(am_ref[0] + am_ref[1], w1b_ref[...])
         + _dot(aw_ref[0] + aw_ref[1], w1c_ref[...])
         + v_ref[0:1, :])
    x = jnp.maximum(x, 0.0)
    x = jnp.maximum(_dot(x, w2_ref[...]) + v_ref[1:2, :], 0.0)
    x = _dot(x, w3_ref[...]) + v_ref[2:3, :]
    o_ref[...] = f + _ln_p(x, mb_ref[...], v_ref[3:4, :], v_ref[4:5, :])


def _node_mlp(p, fl, amp, awp, mb):
    blk = 3128
    npk = NFP // 4
    grid = npk // blk
    mlp = p["mlp"]
    full = lambda s: pl.BlockSpec(s, lambda i: (0, 0))
    w1 = mlp[0]["W"]
    return pl.pallas_call(
        _node_body_p,
        grid=(grid,),
        in_specs=[
            pl.BlockSpec((blk, LP), lambda i: (i, 0)),
            pl.BlockSpec((2, blk, LP), lambda i: (0, i, 0)),
            pl.BlockSpec((2, blk, LP), lambda i: (0, i, 0)),
            full((LP, LP)), full((LP, LP)), full((LP, LP)),
            full((LP, LP)), full((LP, LP)), full((LP, LP)), full((8, LP)),
        ],
        out_specs=pl.BlockSpec((blk, LP), lambda i: (i, 0)),
        out_shape=jax.ShapeDtypeStruct((npk, LP), jnp.float32),
        compiler_params=pltpu.CompilerParams(
            dimension_semantics=("arbitrary",)),
    )(fl, amp, awp, _blk4(w1[0:L]), _blk4(w1[L:2 * L]), _blk4(w1[2 * L:]),
      _blk4(mlp[1]["W"]), _blk4(mlp[2]["W"]), mb, _pack_vecs_p(p))


def _dec_body(x_ref, w1_ref, w2_ref, w3_ref, v_ref, o_ref):
    x = jnp.maximum(_dot(x_ref[...], w1_ref[...]) + v_ref[0:1, :], 0.0)
    x = jnp.maximum(_dot(x, w2_ref[...]) + v_ref[1:2, :], 0.0)
    o_ref[...] = _dot(x, w3_ref[...]) + v_ref[2:3, 0:OUT]


def _decoder(p, fl):
    blk = 3128
    mlp = p["mlp"]
    full = lambda s: pl.BlockSpec(s, lambda i: (0, 0))
    return pl.pallas_call(
        _dec_body,
        grid=(NFP // blk,),
        in_specs=[
            pl.BlockSpec((blk, L), lambda i: (i, 0)),
            full((L, L)), full((L, L)), full((L, OUT)), full((8, L)),
        ],
        out_specs=pl.BlockSpec((blk, OUT), lambda i: (i, 0)),
        out_shape=jax.ShapeDtypeStruct((NFP, OUT), jnp.float32),
        compiler_params=pltpu.CompilerParams(
            dimension_semantics=("arbitrary",)),
    )(fl, mlp[0]["W"], mlp[1]["W"], mlp[2]["W"], _pack_vecs(p, False))


@functools.cache
def _sc_mesh():
    return plsc.VectorSubcoreMesh(
        core_axis_name="c", subcore_axis_name="s",
        num_cores=NC, num_subcores=NS)


NK = 7


def _gather_call(table, idx3, nb):
    rows_pw = nb * BLK
    n = NW * rows_pw
    ng = nb // NK

    @functools.partial(
        pl.kernel,
        out_type=jax.ShapeDtypeStruct((n, L), jnp.float32),
        mesh=_sc_mesh(),
        compiler_params=pltpu.CompilerParams(use_tc_tiling_on_sc=False),
        scratch_types=[
            pltpu.VMEM((nb, BLK), jnp.int32),
            pltpu.VMEM((NK, BLK, L), jnp.float32),
            pltpu.SemaphoreType.DMA,
            pltpu.SemaphoreType.DMA,
        ],
    )
    def k(table_ref, idx_ref, out_ref, idx_v, bufs, semg, sems):
        wid = lax.axis_index("s") * NC + lax.axis_index("c")
        base = wid * rows_pw
        pltpu.sync_copy(idx_ref.at[wid], idx_v)

        def body(g, _):
            b0 = g * NK
            ds = [pltpu.async_copy(table_ref.at[idx_v.at[b0 + j]],
                                   bufs.at[j], semg) for j in range(NK)]
            for d in ds:
                d.wait()
            ss = [pltpu.async_copy(
                bufs.at[j],
                out_ref.at[pl.ds(base + (b0 + j) * BLK, BLK)], sems)
                for j in range(NK)]
            for d in ss:
                d.wait()
            return 0

        lax.fori_loop(0, ng, body, 0)

    return k(table, idx3)


def _scatter_call(vals, idx3, zeros, nb):
    rows_pw = nb * BLK

    @functools.partial(
        pl.kernel,
        out_type=jax.ShapeDtypeStruct((NC, NFP, L), jnp.float32),
        mesh=_sc_mesh(),
        compiler_params=pltpu.CompilerParams(use_tc_tiling_on_sc=False),
        scratch_types=[
            pltpu.VMEM((NK, BLK), jnp.int32),
            pltpu.VMEM((NK, BLK, L), jnp.float32),
            pltpu.VMEM_SHARED((NFP, L), jnp.float32),
            pltpu.SemaphoreType.DMA,
            pltpu.SemaphoreType.DMA,
        ],
    )
    def k(vals_ref, idx_ref, z_ref, out_ref, idx_v, bufs, acc, semr, semw):
        c = lax.axis_index("c")
        s = lax.axis_index("s")
        wid = s * NC + c
        base = wid * rows_pw
        ng = nb // NK
        pltpu.sync_copy(z_ref.at[pl.ds(s * STRIPE, STRIPE)],
                        acc.at[pl.ds(s * STRIPE, STRIPE)])
        plsc.subcore_barrier()

        def body(g, _):
            b0 = g * NK
            di = pltpu.async_copy(idx_ref.at[wid, pl.ds(b0, NK)], idx_v,
                                  semr)
            ds = [pltpu.async_copy(
                vals_ref.at[pl.ds(base + (b0 + j) * BLK, BLK)],
                bufs.at[j], semr) for j in range(NK)]
            di.wait()
            for d in ds:
                d.wait()
            ss = [pltpu.async_copy(bufs.at[j], acc.at[idx_v.at[j]],
                                   semw, add=True) for j in range(NK)]
            for d in ss:
                d.wait()
            return 0

        lax.fori_loop(0, ng, body, 0)
        plsc.subcore_barrier()
        pltpu.sync_copy(acc.at[pl.ds(s * STRIPE, STRIPE)],
                        out_ref.at[c, pl.ds(s * STRIPE, STRIPE)])

    return k(vals, idx3, zeros)


def kernel(fluid_node_attr, env_node_attr, mesh_edge_attr, world_edge_attr,
           params, mesh_edge_index, world_edge_index):
    p = params
    fl_attr = _pad_rows(fluid_node_attr, NFP)
    env_attr = _pad_rows(env_node_attr, NEP)
    me_attr = _pad_rows(mesh_edge_attr, EMP)
    we_attr = _pad_rows(world_edge_attr, EWP)

    ms = _pad_idx(mesh_edge_index[0], EMP, 0)
    md = _pad_idx(mesh_edge_index[1], EMP, NFP - 1)
    ws = _pad_idx(world_edge_index[0], EWP, 0)
    wd = _pad_idx(world_edge_index[1], EWP, NFP - 1)
    gidx = jnp.concatenate([ms, md, wd]).reshape(NW, NB_ALL, BLK)
    ws3 = ws.reshape(NW, NB_W, BLK)
    md3 = md.reshape(NW, NB_M, BLK)
    wd3 = wd.reshape(NW, NB_W, BLK)
    zeros_nf = jnp.zeros((NFP, L), jnp.float32)

    fl = _encoder(p["node_enc"], fl_attr, 3128)
    el = _encoder(p["node_enc"], env_attr, 2512)
    me = _encoder(p["mesh_enc"], me_attr, ROW_T)
    we = _encoder(p["world_enc"], we_attr, ROW_T)

    gws = _gather_call(el, ws3, NB_W)

    pk = lambda x: x.reshape(-1, LP)
    unpk = lambda x: x.reshape(-1, L)
    pk3 = lambda x: x.reshape(2, -1, LP)
    mb = _mean_mat()
    me_p = pk(me)
    we_p = pk(we)
    gws_p = pk(gws)
    nbm = EMP // ROW_T
    for sp in p["steps"]:
        g = _gather_call(fl, gidx, NB_ALL)
        gp = pk(g)
        mnew_p, me_p = _edge_mlp(sp["mesh_edge"], me_p, (gp, 0),
                                 (gp, nbm), mb)
        wnew_p, we_p = _edge_mlp(sp["world_edge"], we_p, (gws_p, 0),
                                 (gp, 2 * nbm), mb)
        amp = _scatter_call(unpk(mnew_p), md3, zeros_nf, NB_M)
        awp = _scatter_call(unpk(wnew_p), wd3, zeros_nf, NB_W)
        fl = unpk(_node_mlp(sp["node"], pk(fl), pk3(amp), pk3(awp), mb))

    return _decoder(p["decoder"], fl)[:NF]


# speedup vs baseline: 6.1626x; 1.0004x over previous
import functools

import jax
import jax.numpy as jnp
from jax import lax
from jax.experimental import pallas as pl
from jax.experimental.pallas import tpu as pltpu
from jax.experimental.pallas import tpu_sc as plsc

L = 32
NF = 50000
NE = 10000
EM = 800000
EW = 200000
OUT = 3

NC = 2
NS = 16
NW = NC * NS
BLK = 128

NB_M = 196
NB_W = 49
EMP = NW * NB_M * BLK
EWP = NW * NB_W * BLK
NB_ALL = 2 * NB_M + NB_W
GM = NW * NB_ALL * BLK

NFP = 50048
NEP = 10048
STRIPE = NFP // NS

ROW_T = 4096


def _pad_rows(x, n):
    return jnp.pad(x, ((0, n - x.shape[0]), (0, 0)))


def _pad_idx(x, n, base, span):
    m = x.shape[0]
    pad = base + jnp.arange(n - m, dtype=jnp.int32) % span
    return jnp.concatenate([x, pad])


def _ln(x, g, b):
    m = jnp.mean(x, axis=-1, keepdims=True)
    v = jnp.mean((x - m) * (x - m), axis=-1, keepdims=True)
    return (x - m) * lax.rsqrt(v + 1e-5) * g + b


def _dot(a, b):
    return jnp.dot(a, b, preferred_element_type=jnp.float32)


def _pack_vecs(p, with_ln):
    mlp = p["mlp"]
    rows = []
    for i in range(3):
        b = mlp[i]["b"]
        rows.append(jnp.pad(b, (0, L - b.shape[0])))
    if with_ln:
        rows.append(p["ln_g"])
        rows.append(p["ln_b"])
    else:
        rows.append(jnp.zeros((L,), jnp.float32))
        rows.append(jnp.zeros((L,), jnp.float32))
    rows += [jnp.zeros((L,), jnp.float32)] * 3
    return jnp.stack(rows)


def _enc_body(x_ref, w1_ref, w2_ref, w3_ref, v_ref, o_ref):
    x = _dot(x_ref[...], w1_ref[...]) + v_ref[0:1, :]
    x = jnp.maximum(x, 0.0)
    x = jnp.maximum(_dot(x, w2_ref[...]) + v_ref[1:2, :], 0.0)
    x = _dot(x, w3_ref[...]) + v_ref[2:3, :]
    o_ref[...] = _ln(x, v_ref[3:4, :], v_ref[4:5, :])


def _encoder(p, x, blk):
    n, f = x.shape
    mlp = p["mlp"]
    full = lambda s: pl.BlockSpec(s, lambda i: (0,) * len(s))
    return pl.pallas_call(
        _enc_body,
        grid=(n // blk,),
        in_specs=[
            pl.BlockSpec((blk, f), lambda i: (i, 0)),
            full((f, L)), full((L, L)), full((L, L)), full((8, L)),
        ],
        out_specs=pl.BlockSpec((blk, L), lambda i: (i, 0)),
        out_shape=jax.ShapeDtypeStruct((n, L), jnp.float32),
        compiler_params=pltpu.CompilerParams(
            dimension_semantics=("arbitrary",)),
    )(x, mlp[0]["W"], mlp[1]["W"], mlp[2]["W"], _pack_vecs(p, True))


LP = 4 * L
RT_P = ROW_T // 4


def _blk4(w):
    return jnp.kron(jnp.eye(4, dtype=jnp.float32), w)


def _mean_mat():
    return _blk4(jnp.full((L, L), 1.0 / L, jnp.float32))


def _pack_vecs_p(p):
    mlp = p["mlp"]
    rows = [jnp.tile(mlp[i]["b"], 4) for i in range(3)]
    rows.append(jnp.tile(p["ln_g"], 4))
    rows.append(jnp.tile(p["ln_b"], 4))
    rows += [jnp.zeros((LP,), jnp.float32)] * 3
    return jnp.stack(rows)


def _ln_p(x, mb, g, b):
    m = _dot(x, mb)
    d = x - m
    v = _dot(d * d, mb)
    return d * lax.rsqrt(v + 1e-5) * g + b


def _edge_body_p(e_ref, s_ref, d_ref, w1a_ref, w1b_ref, w1c_ref,
                 w2_ref, w3_ref, mb_ref, v_ref, new_ref, res_ref):
    e = e_ref[...]
    x = (_dot(e, w1a_ref[...])
         + _dot(s_ref[...], w1b_ref[...])
         + _dot(d_ref[...], w1c_ref[...])
         + v_ref[0:1, :])
    x = jnp.maximum(x, 0.0)
    x = jnp.maximum(_dot(x, w2_ref[...]) + v_ref[1:2, :], 0.0)
    x = _dot(x, w3_ref[...]) + v_ref[2:3, :]
    x = _ln_p(x, mb_ref[...], v_ref[3:4, :], v_ref[4:5, :])
    new_ref[...] = x
    res_ref[...] = e + x


def _edge_mlp(p, e, src, dst, mb):
    n = e.shape[0]
    grid = n // RT_P
    mlp = p["mlp"]
    full = lambda s: pl.BlockSpec(s, lambda i: (0, 0))
    (sa, so), (da, do) = src, dst
    w1 = mlp[0]["W"]
    return pl.pallas_call(
        _edge_body_p,
        grid=(grid,),
        in_specs=[
            pl.BlockSpec((RT_P, LP), lambda i: (i, 0)),
            pl.BlockSpec((RT_P, LP), lambda i, _o=so: (i + _o, 0)),
            pl.BlockSpec((RT_P, LP), lambda i, _o=do: (i + _o, 0)),
            full((LP, LP)), full((LP, LP)), full((LP, LP)),
            full((LP, LP)), full((LP, LP)), full((LP, LP)), full((8, LP)),
        ],
        out_specs=[pl.BlockSpec((RT_P, LP), lambda i: (i, 0))] * 2,
        out_shape=[jax.ShapeDtypeStruct((n, LP), jnp.float32)] * 2,
        compiler_params=pltpu.CompilerParams(
            dimension_semantics=("arbitrary",)),
    )(e, sa, da, _blk4(w1[0:L]), _blk4(w1[L:2 * L]), _blk4(w1[2 * L:]),
      _blk4(mlp[1]["W"]), _blk4(mlp[2]["W"]), mb, _pack_vecs_p(p))


def _node_body_p(f_ref, am_ref, aw_ref, w1a_ref, w1b_ref, w1c_ref,
                 w2_ref, w3_ref, mb_ref, v_ref, o_ref):
    f = f_ref[...]
    x = (_dot(f, w1a_ref[...])
         + _dot(am_ref[0] + am_ref[1], w1b_ref[...])
         + _dot(aw_ref[0] + aw_ref[1], w1c_ref[...])
         + v_ref[0:1, :])
    x = jnp.maximum(x, 0.0)
    x = jnp.maximum(_dot(x, w2_ref[...]) + v_ref[1:2, :], 0.0)
    x = _dot(x, w3_ref[...]) + v_ref[2:3, :]
    o_ref[...] = f + _ln_p(x, mb_ref[...], v_ref[3:4, :], v_ref[4:5, :])


def _node_mlp(p, fl, amp, awp, mb):
    blk = 3128
    npk = NFP // 4
    grid = npk // blk
    mlp = p["mlp"]
    full = lambda s: pl.BlockSpec(s, lambda i: (0, 0))
    w1 = mlp[0]["W"]
    return pl.pallas_call(
        _node_body_p,
        grid=(grid,),
        in_specs=[
            pl.BlockSpec((blk, LP), lambda i: (i, 0)),
            pl.BlockSpec((2, blk, LP), lambda i: (0, i, 0)),
            pl.BlockSpec((2, blk, LP), lambda i: (0, i, 0)),
            full((LP, LP)), full((LP, LP)), full((LP, LP)),
            full((LP, LP)), full((LP, LP)), full((LP, LP)), full((8, LP)),
        ],
        out_specs=pl.BlockSpec((blk, LP), lambda i: (i, 0)),
        out_shape=jax.ShapeDtypeStruct((npk, LP), jnp.float32),
        compiler_params=pltpu.CompilerParams(
            dimension_semantics=("arbitrary",)),
    )(fl, amp, awp, _blk4(w1[0:L]), _blk4(w1[L:2 * L]), _blk4(w1[2 * L:]),
      _blk4(mlp[1]["W"]), _blk4(mlp[2]["W"]), mb, _pack_vecs_p(p))


def _dec_body(x_ref, w1_ref, w2_ref, w3_ref, v_ref, o_ref):
    x = jnp.maximum(_dot(x_ref[...], w1_ref[...]) + v_ref[0:1, :], 0.0)
    x = jnp.maximum(_dot(x, w2_ref[...]) + v_ref[1:2, :], 0.0)
    o_ref[...] = _dot(x, w3_ref[...]) + v_ref[2:3, 0:OUT]


def _decoder(p, fl):
    blk = 3128
    mlp = p["mlp"]
    full = lambda s: pl.BlockSpec(s, lambda i: (0, 0))
    return pl.pallas_call(
        _dec_body,
        grid=(NFP // blk,),
        in_specs=[
            pl.BlockSpec((blk, L), lambda i: (i, 0)),
            full((L, L)), full((L, L)), full((L, OUT)), full((8, L)),
        ],
        out_specs=pl.BlockSpec((blk, OUT), lambda i: (i, 0)),
        out_shape=jax.ShapeDtypeStruct((NFP, OUT), jnp.float32),
        compiler_params=pltpu.CompilerParams(
            dimension_semantics=("arbitrary",)),
    )(fl, mlp[0]["W"], mlp[1]["W"], mlp[2]["W"], _pack_vecs(p, False))


@functools.cache
def _sc_mesh():
    return plsc.VectorSubcoreMesh(
        core_axis_name="c", subcore_axis_name="s",
        num_cores=NC, num_subcores=NS)


NK = 7


def _gather_call(table, idx3, nb):
    rows_pw = nb * BLK
    n = NW * rows_pw
    ng = nb // NK
    trows = table.shape[0]
    tstripe = trows // NS

    @functools.partial(
        pl.kernel,
        out_type=jax.ShapeDtypeStruct((n, L), jnp.float32),
        mesh=_sc_mesh(),
        compiler_params=pltpu.CompilerParams(use_tc_tiling_on_sc=False),
        scratch_types=[
            pltpu.VMEM((NK, BLK), jnp.int32),
            pltpu.VMEM((NK, BLK, L), jnp.float32),
            pltpu.VMEM_SHARED((trows, L), jnp.float32),
            pltpu.SemaphoreType.DMA,
            pltpu.SemaphoreType.DMA,
        ],
    )
    def k(table_ref, idx_ref, out_ref, idx_v, bufs, tab, semg, sems):
        c = lax.axis_index("c")
        s = lax.axis_index("s")
        wid = s * NC + c
        base = wid * rows_pw
        pltpu.sync_copy(table_ref.at[pl.ds(s * tstripe, tstripe)],
                        tab.at[pl.ds(s * tstripe, tstripe)])
        plsc.subcore_barrier()

        def body(g, _):
            b0 = g * NK
            di = pltpu.async_copy(idx_ref.at[wid, pl.ds(b0, NK)], idx_v,
                                  semg)
            di.wait()
            ds = [pltpu.async_copy(tab.at[idx_v.at[j]],
                                   bufs.at[j], semg) for j in range(NK)]
            for d in ds:
                d.wait()
            ss = [pltpu.async_copy(
                bufs.at[j],
                out_ref.at[pl.ds(base + (b0 + j) * BLK, BLK)], sems)
                for j in range(NK)]
            for d in ss:
                d.wait()
            return 0

        lax.fori_loop(0, ng, body, 0)
        plsc.subcore_barrier()

    return k(table, idx3)


def _scatter_call(vals, idx3, zeros, nb):
    rows_pw = nb * BLK

    @functools.partial(
        pl.kernel,
        out_type=jax.ShapeDtypeStruct((NC, NFP, L), jnp.float32),
        mesh=_sc_mesh(),
        compiler_params=pltpu.CompilerParams(use_tc_tiling_on_sc=False),
        scratch_types=[
            pltpu.VMEM((NK, BLK), jnp.int32),
            pltpu.VMEM((NK, BLK, L), jnp.float32),
            pltpu.VMEM_SHARED((NFP, L), jnp.float32),
            pltpu.SemaphoreType.DMA,
            pltpu.SemaphoreType.DMA,
        ],
    )
    def k(vals_ref, idx_ref, z_ref, out_ref, idx_v, bufs, acc, semr, semw):
        c = lax.axis_index("c")
        s = lax.axis_index("s")
        wid = s * NC + c
        base = wid * rows_pw
        ng = nb // NK
        pltpu.sync_copy(z_ref.at[pl.ds(s * STRIPE, STRIPE)],
                        acc.at[pl.ds(s * STRIPE, STRIPE)])
        plsc.subcore_barrier()

        def body(g, _):
            b0 = g * NK
            di = pltpu.async_copy(idx_ref.at[wid, pl.ds(b0, NK)], idx_v,
                                  semr)
            ds = [pltpu.async_copy(
                vals_ref.at[pl.ds(base + (b0 + j) * BLK, BLK)],
                bufs.at[j], semr) for j in range(NK)]
            di.wait()
            for d in ds:
                d.wait()
            ss = [pltpu.async_copy(bufs.at[j], acc.at[idx_v.at[j]],
                                   semw, add=True) for j in range(NK)]
            for d in ss:
                d.wait()
            return 0

        lax.fori_loop(0, ng, body, 0)
        plsc.subcore_barrier()
        pltpu.sync_copy(acc.at[pl.ds(s * STRIPE, STRIPE)],
                        out_ref.at[c, pl.ds(s * STRIPE, STRIPE)])

    return k(vals, idx3, zeros)


def kernel(fluid_node_attr, env_node_attr, mesh_edge_attr, world_edge_attr,
           params, mesh_edge_index, world_edge_index):
    p = params
    fl_attr = _pad_rows(fluid_node_attr, NFP)
    env_attr = _pad_rows(env_node_attr, NEP)
    me_attr = _pad_rows(mesh_edge_attr, EMP)
    we_attr = _pad_rows(world_edge_attr, EWP)

    ms = _pad_idx(mesh_edge_index[0], EMP, 0, NF)
    md = _pad_idx(mesh_edge_index[1], EMP, NF, NFP - NF)
    ws = _pad_idx(world_edge_index[0], EWP, 0, NE)
    wd = _pad_idx(world_edge_index[1], EWP, NF, NFP - NF)
    gidx = jnp.concatenate([ms, md, wd]).reshape(NW, NB_ALL, BLK)
    ws3 = ws.reshape(NW, NB_W, BLK)
    md3 = md.reshape(NW, NB_M, BLK)
    wd3 = wd.reshape(NW, NB_W, BLK)
    zeros_nf = jnp.zeros((NFP, L), jnp.float32)

    fl = _encoder(p["node_enc"], fl_attr, 3128)
    el = _encoder(p["node_enc"], env_attr, 2512)
    me = _encoder(p["mesh_enc"], me_attr, ROW_T)
    we = _encoder(p["world_enc"], we_attr, ROW_T)

    gws = _gather_call(el, ws3, NB_W)

    pk = lambda x: x.reshape(-1, LP)
    unpk = lambda x: x.reshape(-1, L)
    pk3 = lambda x: x.reshape(2, -1, LP)
    mb = _mean_mat()
    me_p = pk(me)
    we_p = pk(we)
    gws_p = pk(gws)
    nbm = EMP // ROW_T
    for sp in p["steps"]:
        g = _gather_call(fl, gidx, NB_ALL)
        gp = pk(g)
        mnew_p, me_p = _edge_mlp(sp["mesh_edge"], me_p, (gp, 0),
                                 (gp, nbm), mb)
        wnew_p, we_p = _edge_mlp(sp["world_edge"], we_p, (gws_p, 0),
                                 (gp, 2 * nbm), mb)
        amp = _scatter_call(unpk(mnew_p), md3, zeros_nf, NB_M)
        awp = _scatter_call(unpk(wnew_p), wd3, zeros_nf, NB_W)
        fl = unpk(_node_mlp(sp["node"], pk(fl), pk3(amp), pk3(awp), mb))

    return _decoder(p["decoder"], fl)[:NF]
